# baseline (speedup 1.0000x reference)
import functools

import jax
import jax.numpy as jnp
from jax import lax
from jax.experimental import pallas as pl
from jax.experimental.pallas import tpu as pltpu
from jax.experimental.pallas import tpu_sc as plsc

NUM_SEG = 1024
D = 128
N_ROWS = 320000
NC = 2
NS = 16
NW = NC * NS
RPW = 8800
SC_ROWS = NW * RPW
TC_ROWS = N_ROWS - SC_ROWS
TC_BLK = 512
TC_NBLK = TC_ROWS // TC_BLK
CHUNK = 80
NCHUNK = RPW // CHUNK
SEG_PER_TILE = NUM_SEG // NS
NBUF = 5
LAG = 2


def _sc_partials(data, ids):
    mesh = plsc.VectorSubcoreMesh(core_axis_name="c", subcore_axis_name="s")

    @functools.partial(
        pl.kernel,
        out_type=jax.ShapeDtypeStruct((NC, NUM_SEG, D), jnp.float32),
        mesh=mesh,
        scratch_types=[
            pltpu.VMEM((NBUF, CHUNK, D), jnp.float32),
            pltpu.VMEM((NCHUNK, CHUNK), jnp.int32),
            pltpu.VMEM((SEG_PER_TILE, D), jnp.float32),
            pltpu.VMEM_SHARED((NUM_SEG, D), jnp.float32),
            [pltpu.SemaphoreType.DMA] * NBUF,
            [pltpu.SemaphoreType.DMA] * NBUF,
        ],
    )
    def body(data_hbm, ids_hbm, out_hbm, rowbuf, idsbuf, zbuf, acc,
             gsems, ssems):
        cid = lax.axis_index("c")
        sid = lax.axis_index("s")
        wid = cid * NS + sid
        base_row = wid * RPW

        def gather(ch, b):
            return pltpu.make_async_copy(
                data_hbm.at[pl.ds(base_row + ch * CHUNK, CHUNK)],
                rowbuf.at[b],
                gsems[b],
            )

        def scatter(ch, b):
            return pltpu.make_async_copy(
                rowbuf.at[b], acc.at[idsbuf.at[ch]], ssems[b])

        for b in range(NBUF):
            gather(b, b).start()
        pltpu.sync_copy(ids_hbm.at[wid], idsbuf)

        zero = jnp.zeros((16,), jnp.float32)

        def zero_body(i, carry):
            for j in range(D // 16):
                zbuf[i, pl.ds(j * 16, 16)] = zero
            return carry

        lax.fori_loop(0, SEG_PER_TILE, zero_body, 0)
        pltpu.sync_copy(zbuf, acc.at[pl.ds(sid * SEG_PER_TILE, SEG_PER_TILE)])
        plsc.subcore_barrier()

        def group_step(g, carry):
            for b in range(NBUF):
                ch = g * NBUF + b
                gather(ch, b).wait()
                scatter(ch, b).start(add=True)
                bo = (b - LAG) % NBUF
                cho = ch - LAG

                @pl.when(cho >= 0)
                def _():
                    scatter(cho, bo).wait()

                    @pl.when(cho + NBUF < NCHUNK)
                    def _():
                        gather(cho + NBUF, bo).start()

            return carry

        lax.fori_loop(0, NCHUNK // NBUF, group_step, 0)
        for t in range(LAG):
            ch = NCHUNK - LAG + t
            scatter(ch, ch % NBUF).wait()
        plsc.subcore_barrier()

        pltpu.sync_copy(
            acc.at[pl.ds(sid * SEG_PER_TILE, SEG_PER_TILE)],
            out_hbm.at[cid].at[pl.ds(sid * SEG_PER_TILE, SEG_PER_TILE)],
        )

    return body(data, ids)


def _tc_body(ids_ref, data_ref, o_ref):
    seg = lax.broadcasted_iota(jnp.int32, (NUM_SEG, TC_BLK), 0)
    onehot = jnp.where(seg == ids_ref[0], 1.0, 0.0).astype(jnp.float32)
    part = jnp.dot(onehot, data_ref[...],
                   preferred_element_type=jnp.float32)

    @pl.when(pl.program_id(0) == 0)
    def _():
        o_ref[...] = jnp.zeros_like(o_ref)

    o_ref[...] += part


_TC_OFF = SC_ROWS // TC_BLK

_tc_partial = pl.pallas_call(
    _tc_body,
    grid=(TC_NBLK,),
    in_specs=[
        pl.BlockSpec((1, 1, TC_BLK), lambda i: (i + _TC_OFF, 0, 0)),
        pl.BlockSpec((TC_BLK, D), lambda i: (i + _TC_OFF, 0)),
    ],
    out_specs=pl.BlockSpec((NUM_SEG, D), lambda i: (0, 0)),
    out_shape=jax.ShapeDtypeStruct((NUM_SEG, D), jnp.float32),
)


def _combine_body(p_ref, t_ref, o_ref):
    o_ref[...] = p_ref[0] + p_ref[1] + t_ref[...]


_combine = pl.pallas_call(
    _combine_body,
    out_shape=jax.ShapeDtypeStruct((NUM_SEG, D), jnp.float32),
)


def kernel(data, segment_ids):
    ids32 = segment_ids.astype(jnp.int32)
    sc_ids = ids32[:SC_ROWS].reshape(NW, NCHUNK, CHUNK)
    tc_ids = ids32.reshape(N_ROWS // TC_BLK, 1, TC_BLK)
    partials = _sc_partials(data, sc_ids)
    tc_part = _tc_partial(tc_ids, data)
    return _combine(partials, tc_part)


# speedup vs baseline: 1.0046x; 1.0046x over previous
import functools

import jax
import jax.numpy as jnp
from jax import lax
from jax.experimental import pallas as pl
from jax.experimental.pallas import tpu as pltpu
from jax.experimental.pallas import tpu_sc as plsc

NUM_SEG = 1024
D = 128
N_ROWS = 320000
NC = 2
NS = 16
NW = NC * NS
RPW = 8400
SC_ROWS = NW * RPW
TC_ROWS = N_ROWS - SC_ROWS
TC_BLK = 512
TC_NBLK = TC_ROWS // TC_BLK
CHUNK = 80
NCHUNK = RPW // CHUNK
SEG_PER_TILE = NUM_SEG // NS
NBUF = 5
LAG = 2


def _sc_partials(data, ids):
    mesh = plsc.VectorSubcoreMesh(core_axis_name="c", subcore_axis_name="s")

    @functools.partial(
        pl.kernel,
        out_type=jax.ShapeDtypeStruct((NC, NUM_SEG, D), jnp.float32),
        mesh=mesh,
        scratch_types=[
            pltpu.VMEM((NBUF, CHUNK, D), jnp.float32),
            pltpu.VMEM((NCHUNK, CHUNK), jnp.int32),
            pltpu.VMEM((SEG_PER_TILE, D), jnp.float32),
            pltpu.VMEM_SHARED((NUM_SEG, D), jnp.float32),
            [pltpu.SemaphoreType.DMA] * NBUF,
            [pltpu.SemaphoreType.DMA] * NBUF,
        ],
    )
    def body(data_hbm, ids_hbm, out_hbm, rowbuf, idsbuf, zbuf, acc,
             gsems, ssems):
        cid = lax.axis_index("c")
        sid = lax.axis_index("s")
        wid = cid * NS + sid
        base_row = wid * RPW

        def gather(ch, b):
            return pltpu.make_async_copy(
                data_hbm.at[pl.ds(base_row + ch * CHUNK, CHUNK)],
                rowbuf.at[b],
                gsems[b],
            )

        def scatter(ch, b):
            return pltpu.make_async_copy(
                rowbuf.at[b], acc.at[idsbuf.at[ch]], ssems[b])

        for b in range(NBUF):
            gather(b, b).start()
        pltpu.sync_copy(ids_hbm.at[wid], idsbuf)

        zero = jnp.zeros((16,), jnp.float32)

        def zero_body(i, carry):
            for j in range(D // 16):
                zbuf[i, pl.ds(j * 16, 16)] = zero
            return carry

        lax.fori_loop(0, SEG_PER_TILE, zero_body, 0)
        pltpu.sync_copy(zbuf, acc.at[pl.ds(sid * SEG_PER_TILE, SEG_PER_TILE)])
        plsc.subcore_barrier()

        def group_step(g, carry):
            for b in range(NBUF):
                ch = g * NBUF + b
                gather(ch, b).wait()
                scatter(ch, b).start(add=True)
                bo = (b - LAG) % NBUF
                cho = ch - LAG

                @pl.when(cho >= 0)
                def _():
                    scatter(cho, bo).wait()

                    @pl.when(cho + NBUF < NCHUNK)
                    def _():
                        gather(cho + NBUF, bo).start()

            return carry

        lax.fori_loop(0, NCHUNK // NBUF, group_step, 0)
        for t in range(LAG):
            ch = NCHUNK - LAG + t
            scatter(ch, ch % NBUF).wait()
        plsc.subcore_barrier()

        pltpu.sync_copy(
            acc.at[pl.ds(sid * SEG_PER_TILE, SEG_PER_TILE)],
            out_hbm.at[cid].at[pl.ds(sid * SEG_PER_TILE, SEG_PER_TILE)],
        )

    return body(data, ids)


def _tc_body(ids_ref, data_ref, o_ref):
    seg = lax.broadcasted_iota(jnp.int32, (NUM_SEG, TC_BLK), 0)
    onehot = jnp.where(seg == ids_ref[0], 1.0, 0.0).astype(jnp.bfloat16)
    d = data_ref[...]
    d_hi = d.astype(jnp.bfloat16)
    d_lo = (d - d_hi.astype(jnp.float32)).astype(jnp.bfloat16)
    part = (jnp.dot(onehot, d_hi, preferred_element_type=jnp.float32)
            + jnp.dot(onehot, d_lo, preferred_element_type=jnp.float32))

    @pl.when(pl.program_id(0) == 0)
    def _():
        o_ref[...] = jnp.zeros_like(o_ref)

    o_ref[...] += part


_TC_OFF = SC_ROWS // TC_BLK

_tc_partial = pl.pallas_call(
    _tc_body,
    grid=(TC_NBLK,),
    in_specs=[
        pl.BlockSpec((1, 1, TC_BLK), lambda i: (i + _TC_OFF, 0, 0)),
        pl.BlockSpec((TC_BLK, D), lambda i: (i + _TC_OFF, 0)),
    ],
    out_specs=pl.BlockSpec((NUM_SEG, D), lambda i: (0, 0)),
    out_shape=jax.ShapeDtypeStruct((NUM_SEG, D), jnp.float32),
)


def _combine_body(p_ref, t_ref, o_ref):
    o_ref[...] = p_ref[0] + p_ref[1] + t_ref[...]


_combine = pl.pallas_call(
    _combine_body,
    out_shape=jax.ShapeDtypeStruct((NUM_SEG, D), jnp.float32),
)


def kernel(data, segment_ids):
    ids32 = segment_ids.astype(jnp.int32)
    sc_ids = ids32[:SC_ROWS].reshape(NW, NCHUNK, CHUNK)
    tc_ids = ids32.reshape(N_ROWS // TC_BLK, 1, TC_BLK)
    partials = _sc_partials(data, sc_ids)
    tc_part = _tc_partial(tc_ids, data)
    return _combine(partials, tc_part)


# speedup vs baseline: 1.1295x; 1.1243x over previous
import functools

import jax
import jax.numpy as jnp
from jax import lax
from jax.experimental import pallas as pl
from jax.experimental.pallas import tpu as pltpu
from jax.experimental.pallas import tpu_sc as plsc

NUM_SEG = 1024
D = 128
N_ROWS = 320000
NC = 2
NS = 16
NW = NC * NS
RPW = 8400
SC_ROWS = NW * RPW
TC_ROWS = N_ROWS - SC_ROWS
TC_BLK = 512
TC_NBLK = TC_ROWS // TC_BLK
CHUNK = 80
NCHUNK = RPW // CHUNK
SEG_PER_TILE = NUM_SEG // NS
NBUF = 5
LAG = 2


def _sc_partials(data, ids):
    mesh = plsc.VectorSubcoreMesh(core_axis_name="c", subcore_axis_name="s")

    @functools.partial(
        pl.kernel,
        out_type=jax.ShapeDtypeStruct((NC, NUM_SEG, D), jnp.float32),
        mesh=mesh,
        scratch_types=[
            pltpu.VMEM((NBUF, CHUNK, D), jnp.float32),
            pltpu.VMEM((NCHUNK, CHUNK), jnp.int32),
            pltpu.VMEM((SEG_PER_TILE, D), jnp.float32),
            pltpu.VMEM_SHARED((NUM_SEG, D), jnp.float32),
            [pltpu.SemaphoreType.DMA] * NBUF,
            [pltpu.SemaphoreType.DMA] * NBUF,
        ],
    )
    def body(data_hbm, ids_hbm, out_hbm, rowbuf, idsbuf, zbuf, acc,
             gsems, ssems):
        cid = lax.axis_index("c")
        sid = lax.axis_index("s")
        wid = cid * NS + sid
        base_row = wid * RPW

        def gather(ch, b):
            return pltpu.make_async_copy(
                data_hbm.at[pl.ds(base_row + ch * CHUNK, CHUNK)],
                rowbuf.at[b],
                gsems[b],
            )

        def scatter(ch, b):
            return pltpu.make_async_copy(
                rowbuf.at[b], acc.at[idsbuf.at[ch]], ssems[b])

        for b in range(NBUF):
            gather(b, b).start()
        pltpu.sync_copy(ids_hbm.at[wid], idsbuf)

        zero = jnp.zeros((16,), jnp.float32)

        def zero_body(i, carry):
            for j in range(D // 16):
                zbuf[i, pl.ds(j * 16, 16)] = zero
            return carry

        lax.fori_loop(0, SEG_PER_TILE, zero_body, 0)
        pltpu.sync_copy(zbuf, acc.at[pl.ds(sid * SEG_PER_TILE, SEG_PER_TILE)])
        plsc.subcore_barrier()

        def group_step(g, carry):
            for b in range(NBUF):
                ch = g * NBUF + b
                gather(ch, b).wait()
                scatter(ch, b).start(add=True)
                bo = (b - LAG) % NBUF
                cho = ch - LAG

                @pl.when(cho >= 0)
                def _():
                    scatter(cho, bo).wait()

                    @pl.when(cho + NBUF < NCHUNK)
                    def _():
                        gather(cho + NBUF, bo).start()

            return carry

        lax.fori_loop(0, NCHUNK // NBUF, group_step, 0)
        for t in range(LAG):
            ch = NCHUNK - LAG + t
            scatter(ch, ch % NBUF).wait()
        plsc.subcore_barrier()

        pltpu.sync_copy(
            acc.at[pl.ds(sid * SEG_PER_TILE, SEG_PER_TILE)],
            out_hbm.at[cid].at[pl.ds(sid * SEG_PER_TILE, SEG_PER_TILE)],
        )

    return body(data, ids)


def _tc_body(ids_ref, data_ref, o_ref):
    seg = lax.broadcasted_iota(jnp.int32, (NUM_SEG, TC_BLK), 0)
    onehot = jnp.where(seg == ids_ref[0], 1.0, 0.0).astype(jnp.float32)
    part = jnp.dot(onehot, data_ref[...],
                   preferred_element_type=jnp.float32)

    @pl.when(pl.program_id(0) == 0)
    def _():
        o_ref[...] = jnp.zeros_like(o_ref)

    o_ref[...] += part


_TC_OFF = SC_ROWS // TC_BLK

_tc_partial = pl.pallas_call(
    _tc_body,
    grid=(TC_NBLK,),
    in_specs=[
        pl.BlockSpec((1, 1, TC_BLK), lambda i: (i + _TC_OFF, 0, 0)),
        pl.BlockSpec((TC_BLK, D), lambda i: (i + _TC_OFF, 0)),
    ],
    out_specs=pl.BlockSpec((NUM_SEG, D), lambda i: (0, 0)),
    out_shape=jax.ShapeDtypeStruct((NUM_SEG, D), jnp.float32),
)


def _combine_body(p_ref, t_ref, o_ref):
    o_ref[...] = p_ref[0] + p_ref[1] + t_ref[...]


_combine = pl.pallas_call(
    _combine_body,
    out_shape=jax.ShapeDtypeStruct((NUM_SEG, D), jnp.float32),
)


def kernel(data, segment_ids):
    ids32 = segment_ids.astype(jnp.int32)
    sc_ids = ids32[:SC_ROWS].reshape(NW, NCHUNK, CHUNK)
    tc_ids = ids32.reshape(N_ROWS // TC_BLK, 1, TC_BLK)
    partials = _sc_partials(data, sc_ids)
    tc_part = _tc_partial(tc_ids, data)
    return _combine(partials, tc_part)


# speedup vs baseline: 1.1332x; 1.0033x over previous
import functools

import jax
import jax.numpy as jnp
from jax import lax
from jax.experimental import pallas as pl
from jax.experimental.pallas import tpu as pltpu
from jax.experimental.pallas import tpu_sc as plsc

NUM_SEG = 1024
D = 128
N_ROWS = 320000
NC = 2
NS = 16
NW = NC * NS
RPW = 8400
SC_ROWS = NW * RPW
TC_ROWS = N_ROWS - SC_ROWS
TC_BLK = 512
TC_NBLK = TC_ROWS // TC_BLK
CHUNK = 120
NCHUNK = RPW // CHUNK
SEG_PER_TILE = NUM_SEG // NS
NBUF = 5
LAG = 2


def _sc_partials(data, ids):
    mesh = plsc.VectorSubcoreMesh(core_axis_name="c", subcore_axis_name="s")

    @functools.partial(
        pl.kernel,
        out_type=jax.ShapeDtypeStruct((NC, NUM_SEG, D), jnp.float32),
        mesh=mesh,
        scratch_types=[
            pltpu.VMEM((NBUF, CHUNK, D), jnp.float32),
            pltpu.VMEM((NCHUNK, CHUNK), jnp.int32),
            pltpu.VMEM((SEG_PER_TILE, D), jnp.float32),
            pltpu.VMEM_SHARED((NUM_SEG, D), jnp.float32),
            [pltpu.SemaphoreType.DMA] * NBUF,
            [pltpu.SemaphoreType.DMA] * NBUF,
        ],
    )
    def body(data_hbm, ids_hbm, out_hbm, rowbuf, idsbuf, zbuf, acc,
             gsems, ssems):
        cid = lax.axis_index("c")
        sid = lax.axis_index("s")
        wid = cid * NS + sid
        base_row = wid * RPW

        def gather(ch, b):
            return pltpu.make_async_copy(
                data_hbm.at[pl.ds(base_row + ch * CHUNK, CHUNK)],
                rowbuf.at[b],
                gsems[b],
            )

        def scatter(ch, b):
            return pltpu.make_async_copy(
                rowbuf.at[b], acc.at[idsbuf.at[ch]], ssems[b])

        for b in range(NBUF):
            gather(b, b).start()
        pltpu.sync_copy(ids_hbm.at[wid], idsbuf)

        zero = jnp.zeros((16,), jnp.float32)

        def zero_body(i, carry):
            for j in range(D // 16):
                zbuf[i, pl.ds(j * 16, 16)] = zero
            return carry

        lax.fori_loop(0, SEG_PER_TILE, zero_body, 0)
        pltpu.sync_copy(zbuf, acc.at[pl.ds(sid * SEG_PER_TILE, SEG_PER_TILE)])
        plsc.subcore_barrier()

        def group_step(g, carry):
            for b in range(NBUF):
                ch = g * NBUF + b
                gather(ch, b).wait()
                scatter(ch, b).start(add=True)
                bo = (b - LAG) % NBUF
                cho = ch - LAG

                @pl.when(cho >= 0)
                def _():
                    scatter(cho, bo).wait()

                    @pl.when(cho + NBUF < NCHUNK)
                    def _():
                        gather(cho + NBUF, bo).start()

            return carry

        lax.fori_loop(0, NCHUNK // NBUF, group_step, 0)
        for t in range(LAG):
            ch = NCHUNK - LAG + t
            scatter(ch, ch % NBUF).wait()
        plsc.subcore_barrier()

        pltpu.sync_copy(
            acc.at[pl.ds(sid * SEG_PER_TILE, SEG_PER_TILE)],
            out_hbm.at[cid].at[pl.ds(sid * SEG_PER_TILE, SEG_PER_TILE)],
        )

    return body(data, ids)


def _tc_body(ids_ref, data_ref, o_ref):
    seg = lax.broadcasted_iota(jnp.int32, (NUM_SEG, TC_BLK), 0)
    onehot = jnp.where(seg == ids_ref[0], 1.0, 0.0).astype(jnp.float32)
    part = jnp.dot(onehot, data_ref[...],
                   preferred_element_type=jnp.float32)

    @pl.when(pl.program_id(0) == 0)
    def _():
        o_ref[...] = jnp.zeros_like(o_ref)

    o_ref[...] += part


_TC_OFF = SC_ROWS // TC_BLK

_tc_partial = pl.pallas_call(
    _tc_body,
    grid=(TC_NBLK,),
    in_specs=[
        pl.BlockSpec((1, 1, TC_BLK), lambda i: (i + _TC_OFF, 0, 0)),
        pl.BlockSpec((TC_BLK, D), lambda i: (i + _TC_OFF, 0)),
    ],
    out_specs=pl.BlockSpec((NUM_SEG, D), lambda i: (0, 0)),
    out_shape=jax.ShapeDtypeStruct((NUM_SEG, D), jnp.float32),
)


def _combine_body(p_ref, t_ref, o_ref):
    o_ref[...] = p_ref[0] + p_ref[1] + t_ref[...]


_combine = pl.pallas_call(
    _combine_body,
    out_shape=jax.ShapeDtypeStruct((NUM_SEG, D), jnp.float32),
)


def kernel(data, segment_ids):
    ids32 = segment_ids.astype(jnp.int32)
    sc_ids = ids32[:SC_ROWS].reshape(NW, NCHUNK, CHUNK)
    tc_ids = ids32.reshape(N_ROWS // TC_BLK, 1, TC_BLK)
    partials = _sc_partials(data, sc_ids)
    tc_part = _tc_partial(tc_ids, data)
    return _combine(partials, tc_part)


# speedup vs baseline: 1.2607x; 1.1125x over previous
import functools

import jax
import jax.numpy as jnp
from jax import lax
from jax.experimental import pallas as pl
from jax.experimental.pallas import tpu as pltpu
from jax.experimental.pallas import tpu_sc as plsc

NUM_SEG = 1024
D = 128
N_ROWS = 320000
NC = 2
NS = 16
NW = NC * NS
RPW = 8400
SC_ROWS = NW * RPW
TC_ROWS = N_ROWS - SC_ROWS
TC_BLK = 640
TC_NBLK = TC_ROWS // TC_BLK
CHUNK = 120
NCHUNK = RPW // CHUNK
SEG_PER_TILE = NUM_SEG // NS
NBUF = 5
LAG = 2


def _sc_partials(data, ids):
    mesh = plsc.VectorSubcoreMesh(core_axis_name="c", subcore_axis_name="s")

    @functools.partial(
        pl.kernel,
        out_type=jax.ShapeDtypeStruct((NC, NUM_SEG, D), jnp.float32),
        mesh=mesh,
        scratch_types=[
            pltpu.VMEM((NBUF, CHUNK, D), jnp.float32),
            pltpu.VMEM((NCHUNK, CHUNK), jnp.int32),
            pltpu.VMEM((SEG_PER_TILE, D), jnp.float32),
            pltpu.VMEM_SHARED((NUM_SEG, D), jnp.float32),
            [pltpu.SemaphoreType.DMA] * NBUF,
            [pltpu.SemaphoreType.DMA] * NBUF,
        ],
    )
    def body(data_hbm, ids_hbm, out_hbm, rowbuf, idsbuf, zbuf, acc,
             gsems, ssems):
        cid = lax.axis_index("c")
        sid = lax.axis_index("s")
        wid = cid * NS + sid
        base_row = wid * RPW

        def gather(ch, b):
            return pltpu.make_async_copy(
                data_hbm.at[pl.ds(base_row + ch * CHUNK, CHUNK)],
                rowbuf.at[b],
                gsems[b],
            )

        def scatter(ch, b):
            return pltpu.make_async_copy(
                rowbuf.at[b], acc.at[idsbuf.at[ch]], ssems[b])

        for b in range(NBUF):
            gather(b, b).start()
        pltpu.sync_copy(ids_hbm.at[wid], idsbuf)

        zero = jnp.zeros((16,), jnp.float32)

        def zero_body(i, carry):
            for j in range(D // 16):
                zbuf[i, pl.ds(j * 16, 16)] = zero
            return carry

        lax.fori_loop(0, SEG_PER_TILE, zero_body, 0)
        pltpu.sync_copy(zbuf, acc.at[pl.ds(sid * SEG_PER_TILE, SEG_PER_TILE)])
        plsc.subcore_barrier()

        def group_step(g, carry):
            for b in range(NBUF):
                ch = g * NBUF + b
                gather(ch, b).wait()
                scatter(ch, b).start(add=True)
                bo = (b - LAG) % NBUF
                cho = ch - LAG

                @pl.when(cho >= 0)
                def _():
                    scatter(cho, bo).wait()

                    @pl.when(cho + NBUF < NCHUNK)
                    def _():
                        gather(cho + NBUF, bo).start()

            return carry

        lax.fori_loop(0, NCHUNK // NBUF, group_step, 0)
        for t in range(LAG):
            ch = NCHUNK - LAG + t
            scatter(ch, ch % NBUF).wait()
        plsc.subcore_barrier()

        pltpu.sync_copy(
            acc.at[pl.ds(sid * SEG_PER_TILE, SEG_PER_TILE)],
            out_hbm.at[cid].at[pl.ds(sid * SEG_PER_TILE, SEG_PER_TILE)],
        )

    return body(data, ids)


def _tc_body(ids_ref, data_ref, o_ref):
    seg = lax.broadcasted_iota(jnp.int32, (NUM_SEG, TC_BLK), 0)
    onehot = jnp.where(seg == ids_ref[0], 1.0, 0.0).astype(jnp.float32)
    part = jnp.dot(onehot, data_ref[...],
                   preferred_element_type=jnp.float32)

    @pl.when(pl.program_id(0) == 0)
    def _():
        o_ref[...] = jnp.zeros_like(o_ref)

    o_ref[...] += part


_TC_OFF = SC_ROWS // TC_BLK

_tc_partial = pl.pallas_call(
    _tc_body,
    grid=(TC_NBLK,),
    in_specs=[
        pl.BlockSpec((1, 1, TC_BLK), lambda i: (i + _TC_OFF, 0, 0)),
        pl.BlockSpec((TC_BLK, D), lambda i: (i + _TC_OFF, 0)),
    ],
    out_specs=pl.BlockSpec((NUM_SEG, D), lambda i: (0, 0)),
    out_shape=jax.ShapeDtypeStruct((NUM_SEG, D), jnp.float32),
)


def _combine_body(p_ref, t_ref, o_ref):
    o_ref[...] = p_ref[0] + p_ref[1] + t_ref[...]


_combine = pl.pallas_call(
    _combine_body,
    out_shape=jax.ShapeDtypeStruct((NUM_SEG, D), jnp.float32),
)


def kernel(data, segment_ids):
    ids32 = segment_ids.astype(jnp.int32)
    sc_ids = ids32[:SC_ROWS].reshape(NW, NCHUNK, CHUNK)
    tc_ids = ids32.reshape(N_ROWS // TC_BLK, 1, TC_BLK)
    partials = _sc_partials(data, sc_ids)
    tc_part = _tc_partial(tc_ids, data)
    return _combine(partials, tc_part)
